# baseline (speedup 1.0000x reference)
import functools

import jax
import jax.numpy as jnp
from jax import lax
from jax.experimental import pallas as pl
from jax.experimental.pallas import tpu as pltpu
from jax.experimental.pallas import tpu_sc as plsc

_BATCH = 4096
_SEQ = 50
_VOCAB = 100000
_D = 128
_NW = 32
_BB_PER_W = _BATCH // _NW
_NCH = 4
_CH = _BB_PER_W // _NCH
_NBUF = 10

_mesh = plsc.VectorSubcoreMesh(core_axis_name="c", subcore_axis_name="s")


@functools.partial(
    pl.kernel,
    mesh=_mesh,
    out_type=jax.ShapeDtypeStruct((_SEQ, _BATCH, _D), jnp.float32),
    scratch_types=[
        pltpu.VMEM((_SEQ, _BB_PER_W), jnp.int32),
        pltpu.VMEM((_NBUF, _CH, _D), jnp.float32),
        pltpu.SemaphoreType.DMA((_NBUF,)),
        pltpu.SemaphoreType.DMA((_NBUF,)),
    ],
)
def _emb_lookup(xt_hbm, table_hbm, out_hbm, idxt_v, rows_v, gsems, ssems):
    wid = lax.axis_index("s") * 2 + lax.axis_index("c")
    b0 = wid * _BB_PER_W
    pltpu.sync_copy(xt_hbm.at[:, pl.ds(b0, _BB_PER_W)], idxt_v)

    def issue_gather(c, b):
        s, half = c // _NCH, c % _NCH
        pltpu.async_copy(
            table_hbm.at[idxt_v.at[s, pl.ds(half * _CH, _CH)]],
            rows_v.at[b],
            gsems.at[b],
        )

    for b in range(_NBUF):
        issue_gather(b, b)

    def group(g, carry):
        i0 = g * _NBUF
        for b in range(_NBUF):
            c = i0 + b
            s, half = c // _NCH, c % _NCH
            pltpu.make_async_copy(
                out_hbm.at[0, pl.ds(0, _CH)], rows_v.at[b], gsems.at[b]
            ).wait()
            pltpu.async_copy(
                rows_v.at[b],
                out_hbm.at[s, pl.ds(b0 + half * _CH, _CH)],
                ssems.at[b],
            )
        for b in range(_NBUF):
            pltpu.make_async_copy(
                rows_v.at[b], out_hbm.at[0, pl.ds(0, _CH)], ssems.at[b]
            ).wait()
            nxt = i0 + _NBUF + b

            @pl.when(nxt < _SEQ * _NCH)
            def _():
                issue_gather(nxt, b)

        return carry

    lax.fori_loop(0, _SEQ * _NCH // _NBUF, group, 0)


def kernel(x, table):
    out = _emb_lookup(x.T, table)
    return jnp.transpose(out, (1, 0, 2))


# speedup vs baseline: 1.0732x; 1.0732x over previous
import functools

import jax
import jax.numpy as jnp
from jax import lax
from jax.experimental import pallas as pl
from jax.experimental.pallas import tpu as pltpu
from jax.experimental.pallas import tpu_sc as plsc

_BATCH = 4096
_SEQ = 50
_VOCAB = 100000
_D = 128
_NW = 32
_BB_PER_W = _BATCH // _NW
_NCH = 2
_CH = _BB_PER_W // _NCH
_NBUF = 10

_mesh = plsc.VectorSubcoreMesh(core_axis_name="c", subcore_axis_name="s")


@functools.partial(
    pl.kernel,
    mesh=_mesh,
    out_type=jax.ShapeDtypeStruct((_SEQ, _BATCH, _D), jnp.float32),
    scratch_types=[
        pltpu.VMEM((_SEQ, _BB_PER_W), jnp.int32),
        pltpu.VMEM((_NBUF, _CH, _D), jnp.float32),
        pltpu.SemaphoreType.DMA((_NBUF,)),
        pltpu.SemaphoreType.DMA((_NBUF,)),
    ],
)
def _emb_lookup(xt_hbm, table_hbm, out_hbm, idxt_v, rows_v, gsems, ssems):
    wid = lax.axis_index("s") * 2 + lax.axis_index("c")
    b0 = wid * _BB_PER_W
    pltpu.sync_copy(xt_hbm.at[:, pl.ds(b0, _BB_PER_W)], idxt_v)

    def issue_gather(c, b):
        s, half = c // _NCH, c % _NCH
        pltpu.async_copy(
            table_hbm.at[idxt_v.at[s, pl.ds(half * _CH, _CH)]],
            rows_v.at[b],
            gsems.at[b],
        )

    for b in range(_NBUF):
        issue_gather(b, b)

    def group(g, carry):
        i0 = g * _NBUF
        for b in range(_NBUF):
            c = i0 + b
            s, half = c // _NCH, c % _NCH
            pltpu.make_async_copy(
                out_hbm.at[0, pl.ds(0, _CH)], rows_v.at[b], gsems.at[b]
            ).wait()
            pltpu.async_copy(
                rows_v.at[b],
                out_hbm.at[s, pl.ds(b0 + half * _CH, _CH)],
                ssems.at[b],
            )
        for b in range(_NBUF):
            pltpu.make_async_copy(
                rows_v.at[b], out_hbm.at[0, pl.ds(0, _CH)], ssems.at[b]
            ).wait()
            nxt = i0 + _NBUF + b

            @pl.when(nxt < _SEQ * _NCH)
            def _():
                issue_gather(nxt, b)

        return carry

    lax.fori_loop(0, _SEQ * _NCH // _NBUF, group, 0)


def kernel(x, table):
    out = _emb_lookup(x.T, table)
    return jnp.transpose(out, (1, 0, 2))
